# Initial kernel scaffold; baseline (speedup 1.0000x reference)
import jax
import jax.numpy as jnp
from jax.experimental import pallas as pl


def kernel(rpn_boxes, rpn_scores):
    raise NotImplementedError("write your pallas kernel here")


# speedup vs baseline: 77.6245x; 77.6245x over previous
import functools

import jax
import jax.numpy as jnp
from jax.experimental import pallas as pl

N = 5000
NP = 5120
R, C = 8, 640
KOUT = 304
NUM_PROPOSALS = 300
IOU_THRESH = 0.7


def _nms_body(planes_ref, brows_ref, valid_ref, conv_ref, out_ref):
    f32 = jnp.float32
    x1 = planes_ref[0]
    y1 = planes_ref[1]
    x2 = planes_ref[2]
    y2 = planes_ref[3]
    area = (x2 - x1) * (y2 - y1)
    row_i = jax.lax.broadcasted_iota(jnp.int32, (R, C), 0)
    col_i = jax.lax.broadcasted_iota(jnp.int32, (R, C), 1)
    idx = row_i * C + col_i

    def cond(carry):
        i, cnt, _ = carry
        return jnp.logical_and(i < NP, cnt < NUM_PROPOSALS)

    def body(carry):
        i, cnt, keep = carry
        m = (idx == i).astype(f32)
        ki = jnp.sum(m * keep)
        bx1 = jnp.sum(m * x1)
        by1 = jnp.sum(m * y1)
        bx2 = jnp.sum(m * x2)
        by2 = jnp.sum(m * y2)
        barea = (bx2 - bx1) * (by2 - by1)
        xx1 = jnp.maximum(bx1, x1)
        yy1 = jnp.maximum(by1, y1)
        xx2 = jnp.minimum(bx2, x2)
        yy2 = jnp.minimum(by2, y2)
        w = jnp.maximum(xx2 - xx1, 0.0)
        h = jnp.maximum(yy2 - yy1, 0.0)
        inter = w * h
        union = jnp.maximum(barea + area - inter, 1e-9)
        iou = inter / union
        sup = (iou > IOU_THRESH).astype(f32) * (idx > i).astype(f32) * ki
        keep = keep * (1.0 - sup)
        return i + 1, cnt + ki, keep

    i0 = jnp.int32(0)
    cnt0 = f32(0.0)
    keep0 = jnp.ones((R, C), f32)
    i_fin, _, keep = jax.lax.while_loop(cond, body, (i0, cnt0, keep0))

    keep = keep * valid_ref[...] * (idx < i_fin).astype(f32)

    cj = jax.lax.broadcasted_iota(jnp.int32, (C, C), 0)
    cl = jax.lax.broadcasted_iota(jnp.int32, (C, C), 1)
    tri = (cj < cl).astype(f32)
    rank_in_row = jnp.dot(keep, tri, preferred_element_type=f32)
    rowsum = jnp.sum(keep, axis=1, keepdims=True)
    r0 = jax.lax.broadcasted_iota(jnp.int32, (R, R), 0)
    r1 = jax.lax.broadcasted_iota(jnp.int32, (R, R), 1)
    tri_r = (r1 < r0).astype(f32)
    offs = jnp.dot(tri_r, rowsum, preferred_element_type=f32)
    rank = rank_in_row + offs

    kio = jax.lax.broadcasted_iota(jnp.int32, (KOUT, C), 0).astype(f32)
    acc = jnp.zeros((KOUT, 4), f32)
    for r in range(R):
        sel = (kio == rank[r:r + 1, :]).astype(f32) * keep[r:r + 1, :]
        acc = acc + jnp.dot(sel, brows_ref[r], preferred_element_type=f32)

    out_ref[...] = jnp.dot(acc, conv_ref[...], preferred_element_type=f32)


@functools.partial(jax.jit, static_argnames=())
def kernel(rpn_boxes, rpn_scores):
    cx, cy = rpn_boxes[:, 0], rpn_boxes[:, 1]
    w, h = rpn_boxes[:, 2], rpn_boxes[:, 3]
    x1 = jnp.clip(cx - w * 0.5, 0.0, 1023.0)
    y1 = jnp.clip(cy - h * 0.5, 0.0, 1023.0)
    x2 = jnp.clip(cx + w * 0.5, 0.0, 1023.0)
    y2 = jnp.clip(cy + h * 0.5, 0.0, 1023.0)
    valid = ((x2 - x1) > 0.0) & ((y2 - y1) > 0.0)
    scores = jax.nn.sigmoid(rpn_scores)
    scores = jnp.where(valid, scores, -1e9)
    order = jnp.argsort(-scores)
    x1s, y1s, x2s, y2s = x1[order], y1[order], x2[order], y2[order]
    vs = (scores[order] > -1e8).astype(jnp.float32)

    pad = NP - N
    def p(a):
        return jnp.concatenate([a, jnp.zeros((pad,), a.dtype)])

    planes = jnp.stack([p(x1s), p(y1s), p(x2s), p(y2s)]).reshape(4, R, C)
    brows = jnp.stack([p(x1s), p(y1s), p(x2s), p(y2s)], axis=1).reshape(R, C, 4)
    validm = p(vs).reshape(R, C)
    conv = jnp.array(
        [[0.5, 0.0, -1.0, 0.0],
         [0.0, 0.5, 0.0, -1.0],
         [0.5, 0.0, 1.0, 0.0],
         [0.0, 0.5, 0.0, 1.0]], jnp.float32)

    out = pl.pallas_call(
        _nms_body,
        out_shape=jax.ShapeDtypeStruct((KOUT, 4), jnp.float32),
    )(planes, brows, validm, conv)
    return out[:NUM_PROPOSALS]
